# baseline (speedup 1.0000x reference)
import jax
import jax.numpy as jnp
from jax.experimental import pallas as pl

_B, _K = 16384, 64
_BLK = 2048


def _body(gu_ref, gi_ref, out_ref):
    out_ref[...] = jnp.sum(gu_ref[...] * gi_ref[...], axis=1, keepdims=True)


def kernel(gu, gi):
    out = pl.pallas_call(
        _body,
        grid=(_B // _BLK,),
        in_specs=[
            pl.BlockSpec((_BLK, _K), lambda i: (i, 0)),
            pl.BlockSpec((_BLK, _K), lambda i: (i, 0)),
        ],
        out_specs=pl.BlockSpec((_BLK, 1), lambda i: (i, 0)),
        out_shape=jax.ShapeDtypeStruct((_B, 1), jnp.float32),
    )(gu, gi)
    return out[:, 0]


# speedup vs baseline: 1.3546x; 1.3546x over previous
import jax
import jax.numpy as jnp
from jax.experimental import pallas as pl

_B, _K = 16384, 64
_BLK = 2048


def _body(gu_ref, gi_ref, out_ref):
    prod = gu_ref[...] * gi_ref[...]
    s = jnp.sum(prod.reshape(_BLK // 128, 128, _K), axis=2)
    out_ref[...] = s


def kernel(gu, gi):
    out = pl.pallas_call(
        _body,
        grid=(_B // _BLK,),
        in_specs=[
            pl.BlockSpec((_BLK, _K), lambda i: (i, 0)),
            pl.BlockSpec((_BLK, _K), lambda i: (i, 0)),
        ],
        out_specs=pl.BlockSpec((_BLK // 128, 128), lambda i: (i, 0)),
        out_shape=jax.ShapeDtypeStruct((_B // 128, 128), jnp.float32),
    )(gu, gi)
    return out.reshape(_B)


# speedup vs baseline: 1.3965x; 1.0309x over previous
import jax
import jax.numpy as jnp
from jax.experimental import pallas as pl

_B, _K = 16384, 64
_BLK = 2048


def _body(gu_ref, gi_ref, out_ref):
    prod = gu_ref[...] * gi_ref[...]
    s = jnp.sum(prod.reshape(_B // 128, 128, _K), axis=2)
    out_ref[...] = s


def kernel(gu, gi):
    out = pl.pallas_call(
        _body,
        out_shape=jax.ShapeDtypeStruct((_B // 128, 128), jnp.float32),
    )(gu, gi)
    return out.reshape(_B)


# speedup vs baseline: 1.5393x; 1.1023x over previous
import jax
import jax.numpy as jnp
from jax.experimental import pallas as pl
from jax.experimental.pallas import tpu as pltpu

_B, _K = 16384, 64
_NCHUNK = 8
_CH = _B // _NCHUNK


def _body(gu_hbm, gi_hbm, out_ref, gu_v, gi_v, sems):
    copies = []
    for c in range(_NCHUNK):
        cu = pltpu.make_async_copy(
            gu_hbm.at[pl.ds(c * _CH, _CH), :],
            gu_v.at[pl.ds(c * _CH, _CH), :],
            sems.at[0, c],
        )
        ci = pltpu.make_async_copy(
            gi_hbm.at[pl.ds(c * _CH, _CH), :],
            gi_v.at[pl.ds(c * _CH, _CH), :],
            sems.at[1, c],
        )
        cu.start()
        ci.start()
        copies.append((cu, ci))
    for c in range(_NCHUNK):
        cu, ci = copies[c]
        cu.wait()
        ci.wait()
        prod = gu_v[pl.ds(c * _CH, _CH), :] * gi_v[pl.ds(c * _CH, _CH), :]
        s = jnp.sum(prod.reshape(_CH // 128, 128, _K), axis=2)
        out_ref[pl.ds(c * (_CH // 128), _CH // 128), :] = s


def kernel(gu, gi):
    out = pl.pallas_call(
        _body,
        in_specs=[
            pl.BlockSpec(memory_space=pltpu.HBM),
            pl.BlockSpec(memory_space=pltpu.HBM),
        ],
        out_specs=pl.BlockSpec(memory_space=pltpu.VMEM),
        out_shape=jax.ShapeDtypeStruct((_B // 128, 128), jnp.float32),
        scratch_shapes=[
            pltpu.VMEM((_B, _K), jnp.float32),
            pltpu.VMEM((_B, _K), jnp.float32),
            pltpu.SemaphoreType.DMA((2, _NCHUNK)),
        ],
    )(gu, gi)
    return out.reshape(_B)
